# baseline (speedup 1.0000x reference)
import jax
import jax.numpy as jnp
from jax import lax
from jax.experimental import pallas as pl
from jax.experimental.pallas import tpu as pltpu
from jax.experimental.pallas import tpu_sc as plsc

N = 10000
NPAD = 10240
E = 320000
D_FEAT = 128
D_EDGE = 16
LATENT = 128
N_ACT = 16

NC = 2
NS = 16
NW = NC * NS
EPW = E // NW
B = 80
NB = EPW // B


def _mm_bias_kernel(x_ref, w_ref, b_ref, o_ref):
    o_ref[...] = (
        jnp.dot(x_ref[...], w_ref[...], preferred_element_type=jnp.float32)
        + b_ref[...]
    )


def _mm_bias(x, W, b, blk):
    M, K = x.shape
    Nout = W.shape[1]
    return pl.pallas_call(
        _mm_bias_kernel,
        grid=(M // blk,),
        in_specs=[
            pl.BlockSpec((blk, K), lambda i: (i, 0)),
            pl.BlockSpec((K, Nout), lambda i: (0, 0)),
            pl.BlockSpec((1, Nout), lambda i: (0, 0)),
        ],
        out_specs=pl.BlockSpec((blk, Nout), lambda i: (i, 0)),
        out_shape=jax.ShapeDtypeStruct((M, Nout), jnp.float32),
    )(x, W, b.reshape(1, -1))


def _combine_mm_kernel(acc_ref, den_ref, bias_ref, w_ref, b2_ref, o_ref):
    den = den_ref[0] + den_ref[1]
    inv = 1.0 / (den + 1e-16)
    lat = (acc_ref[0] + acc_ref[1]) * inv[:, None] + bias_ref[...]
    o_ref[...] = (
        jnp.dot(lat, w_ref[...], preferred_element_type=jnp.float32)
        + b2_ref[...]
    )


def _combine_mm(acc_p, den_p, bias, W, b2, blk=1024):
    D = acc_p.shape[2]
    Nout = W.shape[1]
    return pl.pallas_call(
        _combine_mm_kernel,
        grid=(NPAD // blk,),
        in_specs=[
            pl.BlockSpec((2, blk, D), lambda i: (0, i, 0)),
            pl.BlockSpec((2, blk), lambda i: (0, i)),
            pl.BlockSpec((1, D), lambda i: (0, 0)),
            pl.BlockSpec((D, Nout), lambda i: (0, 0)),
            pl.BlockSpec((1, Nout), lambda i: (0, 0)),
        ],
        out_specs=pl.BlockSpec((blk, Nout), lambda i: (i, 0)),
        out_shape=jax.ShapeDtypeStruct((NPAD, Nout), jnp.float32),
    )(acc_p, den_p, bias.reshape(1, -1), W, b2.reshape(1, -1))


def _final_kernel(acc_ref, den_ref, bias_ref, o_ref):
    den = den_ref[0] + den_ref[1]
    inv = 1.0 / (den + 1e-16)
    o_ref[...] = (acc_ref[0] + acc_ref[1]) * inv[:, None] + bias_ref[...]


def _final_combine(acc_p, den_p, bias, blk=2048):
    D = acc_p.shape[2]
    return pl.pallas_call(
        _final_kernel,
        grid=(NPAD // blk,),
        in_specs=[
            pl.BlockSpec((2, blk, D), lambda i: (0, i, 0)),
            pl.BlockSpec((2, blk), lambda i: (0, i)),
            pl.BlockSpec((1, D), lambda i: (0, 0)),
        ],
        out_specs=pl.BlockSpec((blk, D), lambda i: (i, 0)),
        out_shape=jax.ShapeDtypeStruct((NPAD, D), jnp.float32),
    )(acc_p, den_p, bias.reshape(1, -1))


_SC_MESH = dict(core_axis_name="c", subcore_axis_name="s",
                num_cores=NC, num_subcores=NS)
_SC_PARAMS = pltpu.CompilerParams(
    needs_layout_passes=False, use_tc_tiling_on_sc=False
)


def _run_pipeline(issue_idx, wait_idx, issue_data, wait_data,
                  compute, issue_out, wait_out, S0, S1):
    issue_idx(0, S0)
    wait_idx(S0)
    issue_data(0, S0)
    issue_idx(1, S1)

    def superstep(i, _):
        for k in range(2):
            t = 2 * i + k
            Sp, Sq = (S0, S1) if k == 0 else (S1, S0)
            wait_idx(Sq)
            issue_data(t + 1, Sq)
            wait_data(Sp)

            @pl.when(t >= 2)
            def _():
                wait_out(Sp)
            compute(t, Sp)

            @pl.when(t < NB - 2)
            def _():
                issue_idx(t + 2, Sp)
            issue_out(t, Sp)
        return 0
    lax.fori_loop(0, (NB - 1) // 2, superstep, 0)

    wait_data(S0)
    wait_out(S0)
    compute(NB - 1, S0)
    issue_out(NB - 1, S0)
    wait_out(S1)
    wait_out(S0)


def _make_edge_ex_sc(D):
    mesh = plsc.VectorSubcoreMesh(**_SC_MESH)
    CH = D // 16

    def body(xl_hbm, src_hbm, dst_hbm, ef_hbm, att_hbm, ex_out,
             src0, dst0, sxe0, ex0,
             src1, dst1, sxe1, ex1,
             att_v, sem_i0, sem_i1, sem_g0, sem_g1, sem_w0, sem_w1):
        c = lax.axis_index("c")
        s = lax.axis_index("s")
        g = c * NS + s
        pltpu.sync_copy(att_hbm, att_v)
        iota16 = lax.iota(jnp.int32, 16)

        S0 = (src0, dst0, sxe0, ex0, sem_i0, sem_g0, sem_w0)
        S1 = (src1, dst1, sxe1, ex1, sem_i1, sem_g1, sem_w1)

        def base(t):
            return g * EPW + t * B

        def issue_idx(t, S):
            srcv, dstv, sxev, _, sem_i, _, _ = S
            pltpu.async_copy(src_hbm.at[pl.ds(base(t), B)], srcv, sem_i)
            pltpu.async_copy(dst_hbm.at[pl.ds(base(t), B)], dstv, sem_i)
            pltpu.async_copy(ef_hbm.at[pl.ds(base(t), B)], sxev, sem_i)

        def wait_idx(S):
            srcv, dstv, sxev, _, sem_i, _, _ = S
            pltpu.make_async_copy(src_hbm.at[pl.ds(0, B)], srcv, sem_i).wait()
            pltpu.make_async_copy(dst_hbm.at[pl.ds(0, B)], dstv, sem_i).wait()
            pltpu.make_async_copy(ef_hbm.at[pl.ds(0, B)], sxev, sem_i).wait()

        def issue_data(t, S):
            srcv, dstv, sxev, _, _, sem_g, _ = S
            for j in range(B // 16):
                sl = pl.ds(j * 16, 16)
                pltpu.async_copy(xl_hbm.at[srcv.at[sl]], sxev.at[sl],
                                 sem_g, add=True)
                pltpu.async_copy(xl_hbm.at[dstv.at[sl]], sxev.at[sl],
                                 sem_g, add=True)

        def wait_data(S):
            srcv, dstv, sxev, _, _, sem_g, _ = S
            for j in range(B // 16):
                sl = pl.ds(j * 16, 16)
                pltpu.make_async_copy(
                    xl_hbm.at[srcv.at[sl]], sxev.at[sl], sem_g).wait()
                pltpu.make_async_copy(
                    xl_hbm.at[dstv.at[sl]], sxev.at[sl], sem_g).wait()

        def compute(t, S):
            sxev, exv = S[2], S[3]
            att_c = [att_v[pl.ds(cc * 16, 16)] for cc in range(CH)]

            def group(gi, _):
                def edge(el, lg):
                    e = gi * 16 + el
                    acc0 = jnp.zeros((16,), jnp.float32)
                    acc1 = jnp.zeros((16,), jnp.float32)
                    for cc in range(CH):
                        sv = sxev[e, pl.ds(cc * 16, 16)]
                        lv = jnp.maximum(sv, 0.2 * sv) * att_c[cc]
                        if cc % 2:
                            acc1 = acc1 + lv
                        else:
                            acc0 = acc0 + lv
                    logit = jnp.sum(acc0 + acc1)
                    return jnp.where(iota16 == el, logit, lg)
                lg = lax.fori_loop(0, 16, edge,
                                   jnp.zeros((16,), jnp.float32))
                exv[pl.ds(gi * 16, 16)] = jnp.exp(lg)
                return 0
            lax.fori_loop(0, B // 16, group, 0)

        def issue_out(t, S):
            exv, sem_w = S[3], S[6]
            pltpu.async_copy(exv, ex_out.at[pl.ds(base(t), B)], sem_w)

        def wait_out(S):
            exv, sem_w = S[3], S[6]
            pltpu.make_async_copy(exv, ex_out.at[pl.ds(0, B)], sem_w).wait()

        _run_pipeline(issue_idx, wait_idx, issue_data, wait_data,
                      compute, issue_out, wait_out, S0, S1)

    return pl.kernel(
        body,
        out_type=jax.ShapeDtypeStruct((E,), jnp.float32),
        mesh=mesh,
        compiler_params=_SC_PARAMS,
        scratch_types=[
            pltpu.VMEM((B,), jnp.int32),
            pltpu.VMEM((B,), jnp.int32),
            pltpu.VMEM((B, D), jnp.float32),
            pltpu.VMEM((B,), jnp.float32),
            pltpu.VMEM((B,), jnp.int32),
            pltpu.VMEM((B,), jnp.int32),
            pltpu.VMEM((B, D), jnp.float32),
            pltpu.VMEM((B,), jnp.float32),
            pltpu.VMEM((D,), jnp.float32),
            pltpu.SemaphoreType.DMA,
            pltpu.SemaphoreType.DMA,
            pltpu.SemaphoreType.DMA,
            pltpu.SemaphoreType.DMA,
            pltpu.SemaphoreType.DMA,
            pltpu.SemaphoreType.DMA,
        ],
    )


def _make_scatter_sc(D):
    mesh = plsc.VectorSubcoreMesh(**_SC_MESH)
    CH = D // 16
    RPT = NPAD // NS

    def body(xl_hbm, src_hbm, dst_hbm, ex_hbm,
             den_out, acc_out,
             src0, sdst0, exl0, xs0, w0, sdsc0, exsc0,
             src1, sdst1, exl1, xs1, w1, sdsc1, exsc1,
             zden_v, den_s, acc_s,
             sem_i0, sem_i1, sem_g0, sem_g1, sem_s0, sem_s1):
        c = lax.axis_index("c")
        s = lax.axis_index("s")
        g = c * NS + s
        iota16 = lax.iota(jnp.int32, 16)
        zero16 = jnp.zeros((16,), jnp.float32)

        def zden_body(i, _):
            zden_v[pl.ds(i * 16, 16)] = zero16
            return 0
        lax.fori_loop(0, RPT // 16, zden_body, 0)

        def zrow_body(i, _):
            for cc in range(CH):
                w0[i, pl.ds(cc * 16, 16)] = zero16
            return 0
        lax.fori_loop(0, B, zrow_body, 0)

        pltpu.sync_copy(zden_v, den_s.at[pl.ds(s * RPT, RPT)])
        for j in range(RPT // B):
            pltpu.sync_copy(w0, acc_s.at[pl.ds(s * RPT + j * B, B)])
        plsc.subcore_barrier()

        S0 = (src0, sdst0, exl0, xs0, w0, sdsc0, exsc0, sem_i0, sem_g0, sem_s0)
        S1 = (src1, sdst1, exl1, xs1, w1, sdsc1, exsc1, sem_i1, sem_g1, sem_s1)

        def base(t):
            return g * EPW + t * B

        def issue_idx(t, S):
            srcv, sdstv, exlv = S[0], S[1], S[2]
            sem_i = S[7]
            pltpu.async_copy(src_hbm.at[pl.ds(base(t), B)], srcv, sem_i)
            pltpu.async_copy(dst_hbm.at[pl.ds(base(t), B)], sdstv, sem_i)
            pltpu.async_copy(ex_hbm.at[pl.ds(base(t), B)], exlv, sem_i)

        def wait_idx(S):
            srcv, sdstv, exlv = S[0], S[1], S[2]
            sem_i = S[7]
            pltpu.make_async_copy(src_hbm.at[pl.ds(0, B)], srcv, sem_i).wait()
            pltpu.make_async_copy(dst_hbm.at[pl.ds(0, B)], sdstv, sem_i).wait()
            pltpu.make_async_copy(ex_hbm.at[pl.ds(0, B)], exlv, sem_i).wait()

        def issue_data(t, S):
            srcv, xsv, sem_g = S[0], S[3], S[8]
            for j in range(B // 16):
                sl = pl.ds(j * 16, 16)
                pltpu.async_copy(xl_hbm.at[srcv.at[sl]], xsv.at[sl], sem_g)

        def wait_data(S):
            srcv, xsv, sem_g = S[0], S[3], S[8]
            for j in range(B // 16):
                sl = pl.ds(j * 16, 16)
                pltpu.make_async_copy(
                    xl_hbm.at[srcv.at[sl]], xsv.at[sl], sem_g).wait()

        def compute(t, S):
            sdstv, exlv, xsv, wv, sdscv, exscv = S[1], S[2], S[3], S[4], S[5], S[6]

            def group(gi, _):
                e0 = gi * 16
                ex16 = exlv[pl.ds(e0, 16)]
                sdscv[pl.ds(e0, 16)] = sdstv[pl.ds(e0, 16)]
                exscv[pl.ds(e0, 16)] = ex16

                def edge(eh, _):
                    e = e0 + 2 * eh
                    exa = jnp.take_along_axis(
                        ex16, jnp.full((16,), 2 * eh, jnp.int32), axis=0)
                    exb = jnp.take_along_axis(
                        ex16, jnp.full((16,), 2 * eh + 1, jnp.int32), axis=0)
                    for cc in range(CH):
                        wv[e, pl.ds(cc * 16, 16)] = (
                            xsv[e, pl.ds(cc * 16, 16)] * exa
                        )
                        wv[e + 1, pl.ds(cc * 16, 16)] = (
                            xsv[e + 1, pl.ds(cc * 16, 16)] * exb
                        )
                    return 0
                lax.fori_loop(0, 8, edge, 0)
                return 0
            lax.fori_loop(0, B // 16, group, 0)

        def issue_out(t, S):
            wv, sdscv, exscv, sem_s = S[4], S[5], S[6], S[9]
            pltpu.async_copy(exscv, den_s.at[sdscv], sem_s, add=True)
            pltpu.async_copy(wv, acc_s.at[sdscv], sem_s, add=True)

        def wait_out(S):
            wv, sdscv, exscv, sem_s = S[4], S[5], S[6], S[9]
            pltpu.make_async_copy(exscv, den_s.at[sdscv], sem_s).wait()
            pltpu.make_async_copy(wv, acc_s.at[sdscv], sem_s).wait()

        _run_pipeline(issue_idx, wait_idx, issue_data, wait_data,
                      compute, issue_out, wait_out, S0, S1)

        plsc.subcore_barrier()
        pltpu.sync_copy(den_s.at[pl.ds(s * RPT, RPT)],
                        den_out.at[c, pl.ds(s * RPT, RPT)])
        pltpu.sync_copy(acc_s.at[pl.ds(s * RPT, RPT)],
                        acc_out.at[c, pl.ds(s * RPT, RPT)])

    return pl.kernel(
        body,
        out_type=(
            jax.ShapeDtypeStruct((NC, NPAD), jnp.float32),
            jax.ShapeDtypeStruct((NC, NPAD, D), jnp.float32),
        ),
        mesh=mesh,
        compiler_params=_SC_PARAMS,
        scratch_types=[
            pltpu.VMEM((B,), jnp.int32),
            pltpu.VMEM((B,), jnp.int32),
            pltpu.VMEM((B,), jnp.float32),
            pltpu.VMEM((B, D), jnp.float32),
            pltpu.VMEM((B, D), jnp.float32),
            pltpu.VMEM((B,), jnp.int32),
            pltpu.VMEM((B,), jnp.float32),
            pltpu.VMEM((B,), jnp.int32),
            pltpu.VMEM((B,), jnp.int32),
            pltpu.VMEM((B,), jnp.float32),
            pltpu.VMEM((B, D), jnp.float32),
            pltpu.VMEM((B, D), jnp.float32),
            pltpu.VMEM((B,), jnp.int32),
            pltpu.VMEM((B,), jnp.float32),
            pltpu.VMEM((NPAD // NS,), jnp.float32),
            pltpu.VMEM_SHARED((NPAD,), jnp.float32),
            pltpu.VMEM_SHARED((NPAD, D), jnp.float32),
            pltpu.SemaphoreType.DMA,
            pltpu.SemaphoreType.DMA,
            pltpu.SemaphoreType.DMA,
            pltpu.SemaphoreType.DMA,
            pltpu.SemaphoreType.DMA,
            pltpu.SemaphoreType.DMA,
        ],
    )


_edge_ex_128 = _make_edge_ex_sc(LATENT)
_edge_ex_16 = _make_edge_ex_sc(N_ACT)
_scatter_128 = _make_scatter_sc(LATENT)
_scatter_16 = _make_scatter_sc(N_ACT)


def kernel(x, edge_index, edge_attr,
           W1, b1, We1, att1, bias1,
           W2, b2, We2, att2, bias2):
    src = edge_index[0]
    dst = edge_index[1]
    zero128 = jnp.zeros((LATENT,), jnp.float32)
    zero16 = jnp.zeros((N_ACT,), jnp.float32)

    xl1 = _mm_bias(x, W1, b1, blk=2000)
    ef1 = _mm_bias(edge_attr, We1, zero128, blk=4000)
    ex1 = _edge_ex_128(xl1, src, dst, ef1, att1)
    den1, acc1 = _scatter_128(xl1, src, dst, ex1)

    xl2 = _combine_mm(acc1, den1, bias1, W2, b2)
    ef2 = _mm_bias(edge_attr, We2, zero16, blk=4000)
    ex2 = _edge_ex_16(xl2, src, dst, ef2, att2)
    den2, acc2 = _scatter_16(xl2, src, dst, ex2)

    action_logits = _final_combine(acc2, den2, bias2)[:N]

    flat = action_logits.reshape(-1)
    skey = jax.random.key(42)
    idx = jax.random.categorical(skey, flat)
    log_prob = jax.nn.log_softmax(flat)[idx]
    sel_node, sel_action = jnp.unravel_index(idx, action_logits.shape)
    return (sel_node, sel_action, log_prob)


# speedup vs baseline: 1.2224x; 1.2224x over previous
import jax
import jax.numpy as jnp
from jax import lax
from jax.experimental import pallas as pl
from jax.experimental.pallas import tpu as pltpu
from jax.experimental.pallas import tpu_sc as plsc

N = 10000
NPAD = 10240
E = 320000
D_FEAT = 128
D_EDGE = 16
LATENT = 128
N_ACT = 16

NC = 2
NS = 16
NW = NC * NS
EPW = E // NW
B = 80
NB = EPW // B


def _mm_bias_kernel(x_ref, w_ref, b_ref, o_ref):
    o_ref[...] = (
        jnp.dot(x_ref[...], w_ref[...], preferred_element_type=jnp.float32)
        + b_ref[...]
    )


def _mm_bias(x, W, b, blk):
    M, K = x.shape
    Nout = W.shape[1]
    return pl.pallas_call(
        _mm_bias_kernel,
        grid=(M // blk,),
        in_specs=[
            pl.BlockSpec((blk, K), lambda i: (i, 0)),
            pl.BlockSpec((K, Nout), lambda i: (0, 0)),
            pl.BlockSpec((1, Nout), lambda i: (0, 0)),
        ],
        out_specs=pl.BlockSpec((blk, Nout), lambda i: (i, 0)),
        out_shape=jax.ShapeDtypeStruct((M, Nout), jnp.float32),
    )(x, W, b.reshape(1, -1))


def _combine_mm_kernel(acc_ref, den_ref, bias_ref, w_ref, b2_ref, o_ref):
    den = den_ref[0] + den_ref[1]
    inv = 1.0 / (den + 1e-16)
    lat = (acc_ref[0] + acc_ref[1]) * inv[:, None] + bias_ref[...]
    o_ref[...] = (
        jnp.dot(lat, w_ref[...], preferred_element_type=jnp.float32)
        + b2_ref[...]
    )


def _combine_mm(acc_p, den_p, bias, W, b2, blk=1024):
    D = acc_p.shape[2]
    Nout = W.shape[1]
    return pl.pallas_call(
        _combine_mm_kernel,
        grid=(NPAD // blk,),
        in_specs=[
            pl.BlockSpec((2, blk, D), lambda i: (0, i, 0)),
            pl.BlockSpec((2, blk), lambda i: (0, i)),
            pl.BlockSpec((1, D), lambda i: (0, 0)),
            pl.BlockSpec((D, Nout), lambda i: (0, 0)),
            pl.BlockSpec((1, Nout), lambda i: (0, 0)),
        ],
        out_specs=pl.BlockSpec((blk, Nout), lambda i: (i, 0)),
        out_shape=jax.ShapeDtypeStruct((NPAD, Nout), jnp.float32),
    )(acc_p, den_p, bias.reshape(1, -1), W, b2.reshape(1, -1))


def _final_kernel(acc_ref, den_ref, bias_ref, o_ref):
    den = den_ref[0] + den_ref[1]
    inv = 1.0 / (den + 1e-16)
    o_ref[...] = (acc_ref[0] + acc_ref[1]) * inv[:, None] + bias_ref[...]


def _final_combine(acc_p, den_p, bias, blk=2048):
    D = acc_p.shape[2]
    return pl.pallas_call(
        _final_kernel,
        grid=(NPAD // blk,),
        in_specs=[
            pl.BlockSpec((2, blk, D), lambda i: (0, i, 0)),
            pl.BlockSpec((2, blk), lambda i: (0, i)),
            pl.BlockSpec((1, D), lambda i: (0, 0)),
        ],
        out_specs=pl.BlockSpec((blk, D), lambda i: (i, 0)),
        out_shape=jax.ShapeDtypeStruct((NPAD, D), jnp.float32),
    )(acc_p, den_p, bias.reshape(1, -1))


_SC_MESH = dict(core_axis_name="c", subcore_axis_name="s",
                num_cores=NC, num_subcores=NS)
_SC_PARAMS = pltpu.CompilerParams(
    needs_layout_passes=False, use_tc_tiling_on_sc=False
)


def _run_pipeline(issue_idx, wait_idx, issue_data, wait_data,
                  compute, issue_out, wait_out, S0, S1):
    issue_idx(0, S0)
    wait_idx(S0)
    issue_data(0, S0)
    issue_idx(1, S1)

    def superstep(i, _):
        for k in range(2):
            t = 2 * i + k
            Sp, Sq = (S0, S1) if k == 0 else (S1, S0)
            wait_idx(Sq)
            issue_data(t + 1, Sq)
            wait_data(Sp)

            @pl.when(t >= 2)
            def _():
                wait_out(Sp)
            compute(t, Sp)

            @pl.when(t < NB - 2)
            def _():
                issue_idx(t + 2, Sp)
            issue_out(t, Sp)
        return 0
    lax.fori_loop(0, (NB - 1) // 2, superstep, 0)

    wait_data(S0)
    wait_out(S0)
    compute(NB - 1, S0)
    issue_out(NB - 1, S0)
    wait_out(S1)
    wait_out(S0)


def _make_edge_ex_sc(D):
    mesh = plsc.VectorSubcoreMesh(**_SC_MESH)
    CH = D // 16

    def body(xl_hbm, src_hbm, dst_hbm, ef_hbm, att_hbm, ex_out,
             src0, dst0, sxe0, ex0,
             src1, dst1, sxe1, ex1,
             att_v, sem_i0, sem_i1, sem_g0, sem_g1, sem_w0, sem_w1):
        c = lax.axis_index("c")
        s = lax.axis_index("s")
        g = c * NS + s
        pltpu.sync_copy(att_hbm, att_v)
        iota16 = lax.iota(jnp.int32, 16)

        S0 = (src0, dst0, sxe0, ex0, sem_i0, sem_g0, sem_w0)
        S1 = (src1, dst1, sxe1, ex1, sem_i1, sem_g1, sem_w1)

        def base(t):
            return g * EPW + t * B

        def issue_idx(t, S):
            srcv, dstv, sxev, _, sem_i, _, _ = S
            pltpu.async_copy(src_hbm.at[pl.ds(base(t), B)], srcv, sem_i)
            pltpu.async_copy(dst_hbm.at[pl.ds(base(t), B)], dstv, sem_i)
            pltpu.async_copy(ef_hbm.at[pl.ds(base(t), B)], sxev, sem_i)

        def wait_idx(S):
            srcv, dstv, sxev, _, sem_i, _, _ = S
            pltpu.make_async_copy(src_hbm.at[pl.ds(0, B)], srcv, sem_i).wait()
            pltpu.make_async_copy(dst_hbm.at[pl.ds(0, B)], dstv, sem_i).wait()
            pltpu.make_async_copy(ef_hbm.at[pl.ds(0, B)], sxev, sem_i).wait()

        def issue_data(t, S):
            srcv, dstv, sxev, _, _, sem_g, _ = S
            for j in range(B // 16):
                sl = pl.ds(j * 16, 16)
                pltpu.async_copy(xl_hbm.at[srcv.at[sl]], sxev.at[sl],
                                 sem_g, add=True)
                pltpu.async_copy(xl_hbm.at[dstv.at[sl]], sxev.at[sl],
                                 sem_g, add=True)

        def wait_data(S):
            srcv, dstv, sxev, _, _, sem_g, _ = S
            for j in range(B // 16):
                sl = pl.ds(j * 16, 16)
                pltpu.make_async_copy(
                    xl_hbm.at[srcv.at[sl]], sxev.at[sl], sem_g).wait()
                pltpu.make_async_copy(
                    xl_hbm.at[dstv.at[sl]], sxev.at[sl], sem_g).wait()

        def compute(t, S):
            sxev, exv = S[2], S[3]
            att_c = [att_v[pl.ds(cc * 16, 16)] for cc in range(CH)]

            def group(gi, _):
                def edge(el, lg):
                    e = gi * 16 + el
                    acc0 = jnp.zeros((16,), jnp.float32)
                    acc1 = jnp.zeros((16,), jnp.float32)
                    for cc in range(CH):
                        sv = sxev[e, pl.ds(cc * 16, 16)]
                        lv = jnp.maximum(sv, 0.2 * sv) * att_c[cc]
                        if cc % 2:
                            acc1 = acc1 + lv
                        else:
                            acc0 = acc0 + lv
                    logit = jnp.sum(acc0 + acc1)
                    return jnp.where(iota16 == el, logit, lg)
                lg = lax.fori_loop(0, 16, edge,
                                   jnp.zeros((16,), jnp.float32))
                exv[pl.ds(gi * 16, 16)] = jnp.exp(lg)
                return 0
            lax.fori_loop(0, B // 16, group, 0)

        def issue_out(t, S):
            exv, sem_w = S[3], S[6]
            pltpu.async_copy(exv, ex_out.at[pl.ds(base(t), B)], sem_w)

        def wait_out(S):
            exv, sem_w = S[3], S[6]
            pltpu.make_async_copy(exv, ex_out.at[pl.ds(0, B)], sem_w).wait()

        _run_pipeline(issue_idx, wait_idx, issue_data, wait_data,
                      compute, issue_out, wait_out, S0, S1)

    return pl.kernel(
        body,
        out_type=jax.ShapeDtypeStruct((E,), jnp.float32),
        mesh=mesh,
        compiler_params=_SC_PARAMS,
        scratch_types=[
            pltpu.VMEM((B,), jnp.int32),
            pltpu.VMEM((B,), jnp.int32),
            pltpu.VMEM((B, D), jnp.float32),
            pltpu.VMEM((B,), jnp.float32),
            pltpu.VMEM((B,), jnp.int32),
            pltpu.VMEM((B,), jnp.int32),
            pltpu.VMEM((B, D), jnp.float32),
            pltpu.VMEM((B,), jnp.float32),
            pltpu.VMEM((D,), jnp.float32),
            pltpu.SemaphoreType.DMA,
            pltpu.SemaphoreType.DMA,
            pltpu.SemaphoreType.DMA,
            pltpu.SemaphoreType.DMA,
            pltpu.SemaphoreType.DMA,
            pltpu.SemaphoreType.DMA,
        ],
    )


def _make_scatter_sc(D):
    mesh = plsc.VectorSubcoreMesh(**_SC_MESH)
    CH = D // 16
    RPT = NPAD // NS

    def body(xl_hbm, src_hbm, dst_hbm, ex_hbm,
             den_out, acc_out,
             src0, sdst0, exl0, xs0, w0, sdsc0, exsc0,
             src1, sdst1, exl1, xs1, w1, sdsc1, exsc1,
             zden_v, den_s, acc_s,
             sem_i0, sem_i1, sem_g0, sem_g1, sem_s0, sem_s1):
        c = lax.axis_index("c")
        s = lax.axis_index("s")
        g = c * NS + s
        iota16 = lax.iota(jnp.int32, 16)
        zero16 = jnp.zeros((16,), jnp.float32)

        def zden_body(i, _):
            zden_v[pl.ds(i * 16, 16)] = zero16
            return 0
        lax.fori_loop(0, RPT // 16, zden_body, 0)

        def zrow_body(i, _):
            for cc in range(CH):
                w0[i, pl.ds(cc * 16, 16)] = zero16
            return 0
        lax.fori_loop(0, B, zrow_body, 0)

        pltpu.sync_copy(zden_v, den_s.at[pl.ds(s * RPT, RPT)])
        for j in range(RPT // B):
            pltpu.sync_copy(w0, acc_s.at[pl.ds(s * RPT + j * B, B)])
        plsc.subcore_barrier()

        S0 = (src0, sdst0, exl0, xs0, w0, sdsc0, exsc0, sem_i0, sem_g0, sem_s0)
        S1 = (src1, sdst1, exl1, xs1, w1, sdsc1, exsc1, sem_i1, sem_g1, sem_s1)

        def base(t):
            return g * EPW + t * B

        def issue_idx(t, S):
            srcv, sdstv, exlv = S[0], S[1], S[2]
            sem_i = S[7]
            pltpu.async_copy(src_hbm.at[pl.ds(base(t), B)], srcv, sem_i)
            pltpu.async_copy(dst_hbm.at[pl.ds(base(t), B)], sdstv, sem_i)
            pltpu.async_copy(ex_hbm.at[pl.ds(base(t), B)], exlv, sem_i)

        def wait_idx(S):
            srcv, sdstv, exlv = S[0], S[1], S[2]
            sem_i = S[7]
            pltpu.make_async_copy(src_hbm.at[pl.ds(0, B)], srcv, sem_i).wait()
            pltpu.make_async_copy(dst_hbm.at[pl.ds(0, B)], sdstv, sem_i).wait()
            pltpu.make_async_copy(ex_hbm.at[pl.ds(0, B)], exlv, sem_i).wait()

        def issue_data(t, S):
            srcv, xsv, sem_g = S[0], S[3], S[8]
            for j in range(B // 16):
                sl = pl.ds(j * 16, 16)
                pltpu.async_copy(xl_hbm.at[srcv.at[sl]], xsv.at[sl], sem_g)

        def wait_data(S):
            srcv, xsv, sem_g = S[0], S[3], S[8]
            for j in range(B // 16):
                sl = pl.ds(j * 16, 16)
                pltpu.make_async_copy(
                    xl_hbm.at[srcv.at[sl]], xsv.at[sl], sem_g).wait()

        def compute(t, S):
            sdstv, exlv, xsv, wv, sdscv, exscv = S[1], S[2], S[3], S[4], S[5], S[6]

            def group(gi, _):
                e0 = gi * 16
                ex16 = exlv[pl.ds(e0, 16)]
                sdscv[pl.ds(e0, 16)] = sdstv[pl.ds(e0, 16)]
                exscv[pl.ds(e0, 16)] = ex16

                @plsc.parallel_loop(0, 16, 1, unroll=2)
                def _(el):
                    e = e0 + el
                    exs = jnp.take_along_axis(
                        ex16, jnp.full((16,), el, jnp.int32), axis=0)
                    for cc in range(CH):
                        wv[e, pl.ds(cc * 16, 16)] = (
                            xsv[e, pl.ds(cc * 16, 16)] * exs
                        )
                return 0
            lax.fori_loop(0, B // 16, group, 0)

        def issue_out(t, S):
            wv, sdscv, exscv, sem_s = S[4], S[5], S[6], S[9]
            pltpu.async_copy(exscv, den_s.at[sdscv], sem_s, add=True)
            pltpu.async_copy(wv, acc_s.at[sdscv], sem_s, add=True)

        def wait_out(S):
            wv, sdscv, exscv, sem_s = S[4], S[5], S[6], S[9]
            pltpu.make_async_copy(exscv, den_s.at[sdscv], sem_s).wait()
            pltpu.make_async_copy(wv, acc_s.at[sdscv], sem_s).wait()

        _run_pipeline(issue_idx, wait_idx, issue_data, wait_data,
                      compute, issue_out, wait_out, S0, S1)

        plsc.subcore_barrier()
        pltpu.sync_copy(den_s.at[pl.ds(s * RPT, RPT)],
                        den_out.at[c, pl.ds(s * RPT, RPT)])
        pltpu.sync_copy(acc_s.at[pl.ds(s * RPT, RPT)],
                        acc_out.at[c, pl.ds(s * RPT, RPT)])

    return pl.kernel(
        body,
        out_type=(
            jax.ShapeDtypeStruct((NC, NPAD), jnp.float32),
            jax.ShapeDtypeStruct((NC, NPAD, D), jnp.float32),
        ),
        mesh=mesh,
        compiler_params=_SC_PARAMS,
        scratch_types=[
            pltpu.VMEM((B,), jnp.int32),
            pltpu.VMEM((B,), jnp.int32),
            pltpu.VMEM((B,), jnp.float32),
            pltpu.VMEM((B, D), jnp.float32),
            pltpu.VMEM((B, D), jnp.float32),
            pltpu.VMEM((B,), jnp.int32),
            pltpu.VMEM((B,), jnp.float32),
            pltpu.VMEM((B,), jnp.int32),
            pltpu.VMEM((B,), jnp.int32),
            pltpu.VMEM((B,), jnp.float32),
            pltpu.VMEM((B, D), jnp.float32),
            pltpu.VMEM((B, D), jnp.float32),
            pltpu.VMEM((B,), jnp.int32),
            pltpu.VMEM((B,), jnp.float32),
            pltpu.VMEM((NPAD // NS,), jnp.float32),
            pltpu.VMEM_SHARED((NPAD,), jnp.float32),
            pltpu.VMEM_SHARED((NPAD, D), jnp.float32),
            pltpu.SemaphoreType.DMA,
            pltpu.SemaphoreType.DMA,
            pltpu.SemaphoreType.DMA,
            pltpu.SemaphoreType.DMA,
            pltpu.SemaphoreType.DMA,
            pltpu.SemaphoreType.DMA,
        ],
    )


_edge_ex_128 = _make_edge_ex_sc(LATENT)
_edge_ex_16 = _make_edge_ex_sc(N_ACT)
_scatter_128 = _make_scatter_sc(LATENT)
_scatter_16 = _make_scatter_sc(N_ACT)


def kernel(x, edge_index, edge_attr,
           W1, b1, We1, att1, bias1,
           W2, b2, We2, att2, bias2):
    src = edge_index[0]
    dst = edge_index[1]
    zero128 = jnp.zeros((LATENT,), jnp.float32)
    zero16 = jnp.zeros((N_ACT,), jnp.float32)

    xl1 = _mm_bias(x, W1, b1, blk=2000)
    ef1 = _mm_bias(edge_attr, We1, zero128, blk=4000)
    ex1 = _edge_ex_128(xl1, src, dst, ef1, att1)
    den1, acc1 = _scatter_128(xl1, src, dst, ex1)

    xl2 = _combine_mm(acc1, den1, bias1, W2, b2)
    ef2 = _mm_bias(edge_attr, We2, zero16, blk=4000)
    ex2 = _edge_ex_16(xl2, src, dst, ef2, att2)
    den2, acc2 = _scatter_16(xl2, src, dst, ex2)

    action_logits = _final_combine(acc2, den2, bias2)[:N]

    flat = action_logits.reshape(-1)
    skey = jax.random.key(42)
    idx = jax.random.categorical(skey, flat)
    log_prob = jax.nn.log_softmax(flat)[idx]
    sel_node, sel_action = jnp.unravel_index(idx, action_logits.shape)
    return (sel_node, sel_action, log_prob)


# speedup vs baseline: 1.5070x; 1.2328x over previous
import jax
import jax.numpy as jnp
from jax import lax
from jax.experimental import pallas as pl
from jax.experimental.pallas import tpu as pltpu
from jax.experimental.pallas import tpu_sc as plsc

N = 10000
NPAD = 10240
E = 320000
D_FEAT = 128
D_EDGE = 16
LATENT = 128
N_ACT = 16

NC = 2
NS = 16
NW = NC * NS
EPW = E // NW
B = 80
NB = EPW // B


def _mm_bias_kernel(x_ref, w_ref, b_ref, o_ref):
    o_ref[...] = (
        jnp.dot(x_ref[...], w_ref[...], preferred_element_type=jnp.float32)
        + b_ref[...]
    )


def _mm_bias(x, W, b, blk):
    M, K = x.shape
    Nout = W.shape[1]
    return pl.pallas_call(
        _mm_bias_kernel,
        grid=(M // blk,),
        in_specs=[
            pl.BlockSpec((blk, K), lambda i: (i, 0)),
            pl.BlockSpec((K, Nout), lambda i: (0, 0)),
            pl.BlockSpec((1, Nout), lambda i: (0, 0)),
        ],
        out_specs=pl.BlockSpec((blk, Nout), lambda i: (i, 0)),
        out_shape=jax.ShapeDtypeStruct((M, Nout), jnp.float32),
    )(x, W, b.reshape(1, -1))


def _combine_mm_kernel(acc_ref, den_ref, bias_ref, w_ref, b2_ref, o_ref):
    den = den_ref[0] + den_ref[1]
    inv = 1.0 / (den + 1e-16)
    lat = (acc_ref[0] + acc_ref[1]) * inv[:, None] + bias_ref[...]
    o_ref[...] = (
        jnp.dot(lat, w_ref[...], preferred_element_type=jnp.float32)
        + b2_ref[...]
    )


def _combine_mm(acc_p, den_p, bias, W, b2, blk=1024):
    D = acc_p.shape[2]
    Nout = W.shape[1]
    return pl.pallas_call(
        _combine_mm_kernel,
        grid=(NPAD // blk,),
        in_specs=[
            pl.BlockSpec((2, blk, D), lambda i: (0, i, 0)),
            pl.BlockSpec((2, blk), lambda i: (0, i)),
            pl.BlockSpec((1, D), lambda i: (0, 0)),
            pl.BlockSpec((D, Nout), lambda i: (0, 0)),
            pl.BlockSpec((1, Nout), lambda i: (0, 0)),
        ],
        out_specs=pl.BlockSpec((blk, Nout), lambda i: (i, 0)),
        out_shape=jax.ShapeDtypeStruct((NPAD, Nout), jnp.float32),
    )(acc_p, den_p, bias.reshape(1, -1), W, b2.reshape(1, -1))


def _final_kernel(acc_ref, den_ref, bias_ref, o_ref):
    den = den_ref[0] + den_ref[1]
    inv = 1.0 / (den + 1e-16)
    o_ref[...] = (acc_ref[0] + acc_ref[1]) * inv[:, None] + bias_ref[...]


def _final_combine(acc_p, den_p, bias, blk=2048):
    D = acc_p.shape[2]
    return pl.pallas_call(
        _final_kernel,
        grid=(NPAD // blk,),
        in_specs=[
            pl.BlockSpec((2, blk, D), lambda i: (0, i, 0)),
            pl.BlockSpec((2, blk), lambda i: (0, i)),
            pl.BlockSpec((1, D), lambda i: (0, 0)),
        ],
        out_specs=pl.BlockSpec((blk, D), lambda i: (i, 0)),
        out_shape=jax.ShapeDtypeStruct((NPAD, D), jnp.float32),
    )(acc_p, den_p, bias.reshape(1, -1))


_SC_MESH = dict(core_axis_name="c", subcore_axis_name="s",
                num_cores=NC, num_subcores=NS)
_SC_PARAMS = pltpu.CompilerParams(
    needs_layout_passes=False, use_tc_tiling_on_sc=False
)


def _run_pipeline(issue_idx, wait_idx, issue_data, wait_data,
                  compute, issue_out, wait_out, S0, S1, NBk):
    issue_idx(0, S0)
    wait_idx(S0)
    issue_data(0, S0)
    issue_idx(1, S1)

    def superstep(i, _):
        for k in range(2):
            t = 2 * i + k
            Sp, Sq = (S0, S1) if k == 0 else (S1, S0)
            wait_idx(Sq)
            issue_data(t + 1, Sq)
            wait_data(Sp)

            @pl.when(t >= 2)
            def _():
                wait_out(Sp)
            compute(t, Sp)

            @pl.when(t < NBk - 2)
            def _():
                issue_idx(t + 2, Sp)
            issue_out(t, Sp)
        return 0
    lax.fori_loop(0, (NBk - 1) // 2, superstep, 0)

    wait_data(S0)
    wait_out(S0)
    compute(NBk - 1, S0)
    issue_out(NBk - 1, S0)
    wait_out(S1)
    wait_out(S0)


def _make_edge_ex_sc(D, Bk):
    mesh = plsc.VectorSubcoreMesh(**_SC_MESH)
    CH = D // 16
    NBk = EPW // Bk

    def body(xl_hbm, src_hbm, dst_hbm, ef_hbm, att_hbm, ex_out,
             src0, dst0, sxe0, ex0,
             src1, dst1, sxe1, ex1,
             att_v, sem_i0, sem_i1, sem_g0, sem_g1, sem_w0, sem_w1):
        c = lax.axis_index("c")
        s = lax.axis_index("s")
        g = c * NS + s
        pltpu.sync_copy(att_hbm, att_v)
        iota16 = lax.iota(jnp.int32, 16)

        S0 = (src0, dst0, sxe0, ex0, sem_i0, sem_g0, sem_w0)
        S1 = (src1, dst1, sxe1, ex1, sem_i1, sem_g1, sem_w1)

        def base(t):
            return g * EPW + t * Bk

        def issue_idx(t, S):
            srcv, dstv, sxev, _, sem_i, _, _ = S
            pltpu.async_copy(src_hbm.at[pl.ds(base(t), Bk)], srcv, sem_i)
            pltpu.async_copy(dst_hbm.at[pl.ds(base(t), Bk)], dstv, sem_i)
            pltpu.async_copy(ef_hbm.at[pl.ds(base(t), Bk)], sxev, sem_i)

        def wait_idx(S):
            srcv, dstv, sxev, _, sem_i, _, _ = S
            pltpu.make_async_copy(src_hbm.at[pl.ds(0, Bk)], srcv, sem_i).wait()
            pltpu.make_async_copy(dst_hbm.at[pl.ds(0, Bk)], dstv, sem_i).wait()
            pltpu.make_async_copy(ef_hbm.at[pl.ds(0, Bk)], sxev, sem_i).wait()

        def issue_data(t, S):
            srcv, dstv, sxev, _, _, sem_g, _ = S
            for j in range(Bk // 80):
                sl = pl.ds(j * 80, 80)
                pltpu.async_copy(xl_hbm.at[srcv.at[sl]], sxev.at[sl],
                                 sem_g, add=True)
                pltpu.async_copy(xl_hbm.at[dstv.at[sl]], sxev.at[sl],
                                 sem_g, add=True)

        def wait_data(S):
            srcv, dstv, sxev, _, _, sem_g, _ = S
            for j in range(Bk // 80):
                sl = pl.ds(j * 80, 80)
                pltpu.make_async_copy(
                    xl_hbm.at[srcv.at[sl]], sxev.at[sl], sem_g).wait()
                pltpu.make_async_copy(
                    xl_hbm.at[dstv.at[sl]], sxev.at[sl], sem_g).wait()

        def compute(t, S):
            sxev, exv = S[2], S[3]
            att_c = [att_v[pl.ds(cc * 16, 16)] for cc in range(CH)]

            def group(gi, _):
                def edge(el, lg):
                    e = gi * 16 + el
                    acc0 = jnp.zeros((16,), jnp.float32)
                    acc1 = jnp.zeros((16,), jnp.float32)
                    for cc in range(CH):
                        sv = sxev[e, pl.ds(cc * 16, 16)]
                        lv = jnp.maximum(sv, 0.2 * sv) * att_c[cc]
                        if cc % 2:
                            acc1 = acc1 + lv
                        else:
                            acc0 = acc0 + lv
                    logit = jnp.sum(acc0 + acc1)
                    return jnp.where(iota16 == el, logit, lg)
                lg = lax.fori_loop(0, 16, edge,
                                   jnp.zeros((16,), jnp.float32))
                exv[pl.ds(gi * 16, 16)] = jnp.exp(lg)
                return 0
            lax.fori_loop(0, Bk // 16, group, 0)

        def issue_out(t, S):
            exv, sem_w = S[3], S[6]
            pltpu.async_copy(exv, ex_out.at[pl.ds(base(t), Bk)], sem_w)

        def wait_out(S):
            exv, sem_w = S[3], S[6]
            pltpu.make_async_copy(exv, ex_out.at[pl.ds(0, Bk)], sem_w).wait()

        _run_pipeline(issue_idx, wait_idx, issue_data, wait_data,
                      compute, issue_out, wait_out, S0, S1, NBk)

    return pl.kernel(
        body,
        out_type=jax.ShapeDtypeStruct((E,), jnp.float32),
        mesh=mesh,
        compiler_params=_SC_PARAMS,
        scratch_types=[
            pltpu.VMEM((Bk,), jnp.int32),
            pltpu.VMEM((Bk,), jnp.int32),
            pltpu.VMEM((Bk, D), jnp.float32),
            pltpu.VMEM((Bk,), jnp.float32),
            pltpu.VMEM((Bk,), jnp.int32),
            pltpu.VMEM((Bk,), jnp.int32),
            pltpu.VMEM((Bk, D), jnp.float32),
            pltpu.VMEM((Bk,), jnp.float32),
            pltpu.VMEM((D,), jnp.float32),
            pltpu.SemaphoreType.DMA,
            pltpu.SemaphoreType.DMA,
            pltpu.SemaphoreType.DMA,
            pltpu.SemaphoreType.DMA,
            pltpu.SemaphoreType.DMA,
            pltpu.SemaphoreType.DMA,
        ],
    )


def _make_scatter_sc(D):
    mesh = plsc.VectorSubcoreMesh(**_SC_MESH)
    CH = D // 16
    RPT = NPAD // NS

    def body(xl_hbm, src_hbm, dst_hbm, ex_hbm,
             den_out, acc_out,
             src0, sdst0, exl0, xs0, w0, sdsc0, exsc0,
             src1, sdst1, exl1, xs1, w1, sdsc1, exsc1,
             zden_v, den_s, acc_s,
             sem_i0, sem_i1, sem_g0, sem_g1, sem_s0, sem_s1):
        c = lax.axis_index("c")
        s = lax.axis_index("s")
        g = c * NS + s
        iota16 = lax.iota(jnp.int32, 16)
        zero16 = jnp.zeros((16,), jnp.float32)

        def zden_body(i, _):
            zden_v[pl.ds(i * 16, 16)] = zero16
            return 0
        lax.fori_loop(0, RPT // 16, zden_body, 0)

        def zrow_body(i, _):
            for cc in range(CH):
                w0[i, pl.ds(cc * 16, 16)] = zero16
            return 0
        lax.fori_loop(0, B, zrow_body, 0)

        pltpu.sync_copy(zden_v, den_s.at[pl.ds(s * RPT, RPT)])
        for j in range(RPT // B):
            pltpu.sync_copy(w0, acc_s.at[pl.ds(s * RPT + j * B, B)])
        plsc.subcore_barrier()

        S0 = (src0, sdst0, exl0, xs0, w0, sdsc0, exsc0, sem_i0, sem_g0, sem_s0)
        S1 = (src1, sdst1, exl1, xs1, w1, sdsc1, exsc1, sem_i1, sem_g1, sem_s1)

        def base(t):
            return g * EPW + t * B

        def issue_idx(t, S):
            srcv, sdstv, exlv = S[0], S[1], S[2]
            sem_i = S[7]
            pltpu.async_copy(src_hbm.at[pl.ds(base(t), B)], srcv, sem_i)
            pltpu.async_copy(dst_hbm.at[pl.ds(base(t), B)], sdstv, sem_i)
            pltpu.async_copy(ex_hbm.at[pl.ds(base(t), B)], exlv, sem_i)

        def wait_idx(S):
            srcv, sdstv, exlv = S[0], S[1], S[2]
            sem_i = S[7]
            pltpu.make_async_copy(src_hbm.at[pl.ds(0, B)], srcv, sem_i).wait()
            pltpu.make_async_copy(dst_hbm.at[pl.ds(0, B)], sdstv, sem_i).wait()
            pltpu.make_async_copy(ex_hbm.at[pl.ds(0, B)], exlv, sem_i).wait()

        def issue_data(t, S):
            srcv, xsv, sem_g = S[0], S[3], S[8]
            for j in range(B // 16):
                sl = pl.ds(j * 16, 16)
                pltpu.async_copy(xl_hbm.at[srcv.at[sl]], xsv.at[sl], sem_g)

        def wait_data(S):
            srcv, xsv, sem_g = S[0], S[3], S[8]
            for j in range(B // 16):
                sl = pl.ds(j * 16, 16)
                pltpu.make_async_copy(
                    xl_hbm.at[srcv.at[sl]], xsv.at[sl], sem_g).wait()

        def compute(t, S):
            sdstv, exlv, xsv, wv, sdscv, exscv = S[1], S[2], S[3], S[4], S[5], S[6]

            def group(gi, _):
                e0 = gi * 16
                ex16 = exlv[pl.ds(e0, 16)]
                sdscv[pl.ds(e0, 16)] = sdstv[pl.ds(e0, 16)]
                exscv[pl.ds(e0, 16)] = ex16

                @plsc.parallel_loop(0, 16, 1, unroll=2)
                def _(el):
                    e = e0 + el
                    exs = jnp.take_along_axis(
                        ex16, jnp.full((16,), el, jnp.int32), axis=0)
                    for cc in range(CH):
                        wv[e, pl.ds(cc * 16, 16)] = (
                            xsv[e, pl.ds(cc * 16, 16)] * exs
                        )
                return 0
            lax.fori_loop(0, B // 16, group, 0)

        def issue_out(t, S):
            wv, sdscv, exscv, sem_s = S[4], S[5], S[6], S[9]
            pltpu.async_copy(exscv, den_s.at[sdscv], sem_s, add=True)
            pltpu.async_copy(wv, acc_s.at[sdscv], sem_s, add=True)

        def wait_out(S):
            wv, sdscv, exscv, sem_s = S[4], S[5], S[6], S[9]
            pltpu.make_async_copy(exscv, den_s.at[sdscv], sem_s).wait()
            pltpu.make_async_copy(wv, acc_s.at[sdscv], sem_s).wait()

        _run_pipeline(issue_idx, wait_idx, issue_data, wait_data,
                      compute, issue_out, wait_out, S0, S1, NB)

        plsc.subcore_barrier()
        pltpu.sync_copy(den_s.at[pl.ds(s * RPT, RPT)],
                        den_out.at[c, pl.ds(s * RPT, RPT)])
        pltpu.sync_copy(acc_s.at[pl.ds(s * RPT, RPT)],
                        acc_out.at[c, pl.ds(s * RPT, RPT)])

    return pl.kernel(
        body,
        out_type=(
            jax.ShapeDtypeStruct((NC, NPAD), jnp.float32),
            jax.ShapeDtypeStruct((NC, NPAD, D), jnp.float32),
        ),
        mesh=mesh,
        compiler_params=_SC_PARAMS,
        scratch_types=[
            pltpu.VMEM((B,), jnp.int32),
            pltpu.VMEM((B,), jnp.int32),
            pltpu.VMEM((B,), jnp.float32),
            pltpu.VMEM((B, D), jnp.float32),
            pltpu.VMEM((B, D), jnp.float32),
            pltpu.VMEM((B,), jnp.int32),
            pltpu.VMEM((B,), jnp.float32),
            pltpu.VMEM((B,), jnp.int32),
            pltpu.VMEM((B,), jnp.int32),
            pltpu.VMEM((B,), jnp.float32),
            pltpu.VMEM((B, D), jnp.float32),
            pltpu.VMEM((B, D), jnp.float32),
            pltpu.VMEM((B,), jnp.int32),
            pltpu.VMEM((B,), jnp.float32),
            pltpu.VMEM((NPAD // NS,), jnp.float32),
            pltpu.VMEM_SHARED((NPAD,), jnp.float32),
            pltpu.VMEM_SHARED((NPAD, D), jnp.float32),
            pltpu.SemaphoreType.DMA,
            pltpu.SemaphoreType.DMA,
            pltpu.SemaphoreType.DMA,
            pltpu.SemaphoreType.DMA,
            pltpu.SemaphoreType.DMA,
            pltpu.SemaphoreType.DMA,
        ],
    )


def _make_fused_sc(D, Bk):
    mesh = plsc.VectorSubcoreMesh(**_SC_MESH)
    CH = D // 16
    NBk = EPW // Bk
    RPT = NPAD // NS

    def body(xl_hbm, src_hbm, dst_hbm, ef_hbm, att_hbm,
             den_out, acc_out,
             src0, dst0, sxd0, xs0, w0, sdsc0, exsc0,
             src1, dst1, sxd1, xs1, w1, sdsc1, exsc1,
             att_v, zden_v, den_s, acc_s,
             sem_i0, sem_i1, sem_g0, sem_g1, sem_s0, sem_s1):
        c = lax.axis_index("c")
        s = lax.axis_index("s")
        g = c * NS + s
        iota16 = lax.iota(jnp.int32, 16)
        zero16 = jnp.zeros((16,), jnp.float32)

        def zden_body(i, _):
            zden_v[pl.ds(i * 16, 16)] = zero16
            return 0
        lax.fori_loop(0, RPT // 16, zden_body, 0)

        def zrow_body(i, _):
            for cc in range(CH):
                w0[i, pl.ds(cc * 16, 16)] = zero16
            return 0
        lax.fori_loop(0, Bk, zrow_body, 0)

        pltpu.sync_copy(zden_v, den_s.at[pl.ds(s * RPT, RPT)])
        pltpu.sync_copy(w0, acc_s.at[pl.ds(s * RPT, Bk)])
        pltpu.sync_copy(w0.at[pl.ds(0, RPT - Bk)],
                        acc_s.at[pl.ds(s * RPT + Bk, RPT - Bk)])
        plsc.subcore_barrier()
        pltpu.sync_copy(att_hbm, att_v)

        S0 = (src0, dst0, sxd0, xs0, w0, sdsc0, exsc0,
              sem_i0, sem_g0, sem_s0)
        S1 = (src1, dst1, sxd1, xs1, w1, sdsc1, exsc1,
              sem_i1, sem_g1, sem_s1)

        def base(t):
            return g * EPW + t * Bk

        def issue_idx(t, S):
            srcv, dstv, sxdv, sem_i = S[0], S[1], S[2], S[7]
            pltpu.async_copy(src_hbm.at[pl.ds(base(t), Bk)], srcv, sem_i)
            pltpu.async_copy(dst_hbm.at[pl.ds(base(t), Bk)], dstv, sem_i)
            pltpu.async_copy(ef_hbm.at[pl.ds(base(t), Bk)], sxdv, sem_i)

        def wait_idx(S):
            srcv, dstv, sxdv, sem_i = S[0], S[1], S[2], S[7]
            pltpu.make_async_copy(src_hbm.at[pl.ds(0, Bk)], srcv, sem_i).wait()
            pltpu.make_async_copy(dst_hbm.at[pl.ds(0, Bk)], dstv, sem_i).wait()
            pltpu.make_async_copy(ef_hbm.at[pl.ds(0, Bk)], sxdv, sem_i).wait()

        def issue_data(t, S):
            srcv, dstv, sxdv, xsv, sem_g = S[0], S[1], S[2], S[3], S[8]
            for j in range(Bk // 80):
                sl = pl.ds(j * 80, 80)
                pltpu.async_copy(xl_hbm.at[dstv.at[sl]], sxdv.at[sl],
                                 sem_g, add=True)
                pltpu.async_copy(xl_hbm.at[srcv.at[sl]], xsv.at[sl], sem_g)

        def wait_data(S):
            srcv, dstv, sxdv, xsv, sem_g = S[0], S[1], S[2], S[3], S[8]
            for j in range(Bk // 80):
                sl = pl.ds(j * 80, 80)
                pltpu.make_async_copy(
                    xl_hbm.at[dstv.at[sl]], sxdv.at[sl], sem_g).wait()
                pltpu.make_async_copy(
                    xl_hbm.at[srcv.at[sl]], xsv.at[sl], sem_g).wait()

        def compute(t, S):
            dstv, sxdv, xsv, wv, sdscv, exscv = (
                S[1], S[2], S[3], S[4], S[5], S[6])
            att_c = [att_v[pl.ds(cc * 16, 16)] for cc in range(CH)]

            def group(gi, _):
                e0 = gi * 16
                sdscv[pl.ds(e0, 16)] = dstv[pl.ds(e0, 16)]

                def edge(el, lg):
                    e = e0 + el
                    acc0 = jnp.zeros((16,), jnp.float32)
                    for cc in range(CH):
                        sv = (sxdv[e, pl.ds(cc * 16, 16)]
                              + xsv[e, pl.ds(cc * 16, 16)])
                        acc0 = acc0 + jnp.maximum(sv, 0.2 * sv) * att_c[cc]
                    logit = jnp.sum(acc0)
                    return jnp.where(iota16 == el, logit, lg)
                lg = lax.fori_loop(0, 16, edge,
                                   jnp.zeros((16,), jnp.float32))
                ex16 = jnp.exp(lg)
                exscv[pl.ds(e0, 16)] = ex16

                @plsc.parallel_loop(0, 16, 1, unroll=2)
                def _(el):
                    e = e0 + el
                    exs = jnp.take_along_axis(
                        ex16, jnp.full((16,), el, jnp.int32), axis=0)
                    for cc in range(CH):
                        wv[e, pl.ds(cc * 16, 16)] = (
                            xsv[e, pl.ds(cc * 16, 16)] * exs
                        )
                return 0
            lax.fori_loop(0, Bk // 16, group, 0)

        def issue_out(t, S):
            wv, sdscv, exscv, sem_s = S[4], S[5], S[6], S[9]
            pltpu.async_copy(exscv, den_s.at[sdscv], sem_s, add=True)
            pltpu.async_copy(wv, acc_s.at[sdscv], sem_s, add=True)

        def wait_out(S):
            wv, sdscv, exscv, sem_s = S[4], S[5], S[6], S[9]
            pltpu.make_async_copy(exscv, den_s.at[sdscv], sem_s).wait()
            pltpu.make_async_copy(wv, acc_s.at[sdscv], sem_s).wait()

        _run_pipeline(issue_idx, wait_idx, issue_data, wait_data,
                      compute, issue_out, wait_out, S0, S1, NBk)

        plsc.subcore_barrier()
        pltpu.sync_copy(den_s.at[pl.ds(s * RPT, RPT)],
                        den_out.at[c, pl.ds(s * RPT, RPT)])
        pltpu.sync_copy(acc_s.at[pl.ds(s * RPT, RPT)],
                        acc_out.at[c, pl.ds(s * RPT, RPT)])

    def _set_scratch():
        return [
            pltpu.VMEM((Bk,), jnp.int32),
            pltpu.VMEM((Bk,), jnp.int32),
            pltpu.VMEM((Bk, D), jnp.float32),
            pltpu.VMEM((Bk, D), jnp.float32),
            pltpu.VMEM((Bk, D), jnp.float32),
            pltpu.VMEM((Bk,), jnp.int32),
            pltpu.VMEM((Bk,), jnp.float32),
        ]

    return pl.kernel(
        body,
        out_type=(
            jax.ShapeDtypeStruct((NC, NPAD), jnp.float32),
            jax.ShapeDtypeStruct((NC, NPAD, D), jnp.float32),
        ),
        mesh=mesh,
        compiler_params=_SC_PARAMS,
        scratch_types=(
            _set_scratch() + _set_scratch()
            + [
                pltpu.VMEM((D,), jnp.float32),
                pltpu.VMEM((NPAD // NS,), jnp.float32),
                pltpu.VMEM_SHARED((NPAD,), jnp.float32),
                pltpu.VMEM_SHARED((NPAD, D), jnp.float32),
                pltpu.SemaphoreType.DMA,
                pltpu.SemaphoreType.DMA,
                pltpu.SemaphoreType.DMA,
                pltpu.SemaphoreType.DMA,
                pltpu.SemaphoreType.DMA,
                pltpu.SemaphoreType.DMA,
            ]
        ),
    )


_edge_ex_128 = _make_edge_ex_sc(LATENT, 400)
_scatter_128 = _make_scatter_sc(LATENT)
_fused_16 = _make_fused_sc(N_ACT, 400)


def kernel(x, edge_index, edge_attr,
           W1, b1, We1, att1, bias1,
           W2, b2, We2, att2, bias2):
    src = edge_index[0]
    dst = edge_index[1]
    zero128 = jnp.zeros((LATENT,), jnp.float32)
    zero16 = jnp.zeros((N_ACT,), jnp.float32)

    xl1 = _mm_bias(x, W1, b1, blk=2000)
    ef1 = _mm_bias(edge_attr, We1, zero128, blk=4000)
    ex1 = _edge_ex_128(xl1, src, dst, ef1, att1)
    den1, acc1 = _scatter_128(xl1, src, dst, ex1)

    xl2 = _combine_mm(acc1, den1, bias1, W2, b2)
    ef2 = _mm_bias(edge_attr, We2, zero16, blk=4000)
    den2, acc2 = _fused_16(xl2, src, dst, ef2, att2)

    action_logits = _final_combine(acc2, den2, bias2)[:N]

    flat = action_logits.reshape(-1)
    skey = jax.random.key(42)
    idx = jax.random.categorical(skey, flat)
    log_prob = jax.nn.log_softmax(flat)[idx]
    sel_node, sel_action = jnp.unravel_index(idx, action_logits.shape)
    return (sel_node, sel_action, log_prob)
